# Initial kernel scaffold; baseline (speedup 1.0000x reference)
import jax
import jax.numpy as jnp
from jax.experimental import pallas as pl


def kernel(x, edge_index, W1, b1, W2, b2):
    raise NotImplementedError("write your pallas kernel here")


# speedup vs baseline: 13.9155x; 13.9155x over previous
import functools

import jax
import jax.numpy as jnp
from jax import lax
from jax.experimental import pallas as pl
from jax.experimental.pallas import tpu as pltpu
from jax.experimental.pallas import tpu_sc as plsc

N = 100000
NPAD = 102400
E = 1600000
FLAT = 2 * NPAD
SLICE = FLAT // 16
IDXBUF = 2048
Q = 49
TOT = 32 * Q * IDXBUF
FM_ROWS = NPAD // 128
TC_BLK = 40
TC_GRID = FM_ROWS // TC_BLK

_mesh = plsc.VectorSubcoreMesh(
    core_axis_name="c", subcore_axis_name="s", num_cores=2, num_subcores=16
)


@functools.partial(
    pl.kernel,
    out_type=jax.ShapeDtypeStruct((2, FLAT), jnp.float32),
    mesh=_mesh,
    scratch_types=[
        pltpu.VMEM_SHARED((FLAT,), jnp.float32),
        pltpu.VMEM_SHARED((FLAT,), jnp.float32),
        pltpu.VMEM((IDXBUF,), jnp.int32),
        pltpu.VMEM((IDXBUF,), jnp.int32),
        pltpu.VMEM((IDXBUF,), jnp.float32),
        pltpu.SemaphoreType.DMA,
    ],
)
def _sc_agg(xs_hbm, src_hbm, dst_hbm, zeros_hbm, out_hbm,
            xs_sp, acc_sp, src_v, dst_v, vals_v, sem):
    cid = lax.axis_index("c")
    sid = lax.axis_index("s")
    r0 = sid * SLICE
    w = cid * 16 + sid
    pltpu.sync_copy(xs_hbm.at[pl.ds(r0, SLICE)], xs_sp.at[pl.ds(r0, SLICE)])
    pltpu.sync_copy(zeros_hbm.at[pl.ds(r0, SLICE)], acc_sp.at[pl.ds(r0, SLICE)])
    plsc.subcore_barrier()

    def body(q, carry):
        off = (w * Q + q) * IDXBUF
        pltpu.sync_copy(src_hbm.at[pl.ds(off, IDXBUF)], src_v)
        pltpu.sync_copy(dst_hbm.at[pl.ds(off, IDXBUF)], dst_v)
        pltpu.async_copy(xs_sp.at[src_v], vals_v, sem).wait()
        pltpu.sync_copy(vals_v, acc_sp.at[dst_v], add=True)
        return carry

    lax.fori_loop(0, Q, body, 0)
    plsc.subcore_barrier()
    pltpu.sync_copy(acc_sp.at[pl.ds(r0, SLICE)], out_hbm.at[cid, pl.ds(r0, SLICE)])


def _tc_a_body(degp_ref, x_ref, dinv_ref, xs1_ref):
    deg = degp_ref[0] + degp_ref[1] + 1.0
    dinv = lax.rsqrt(deg)
    dinv_ref[...] = dinv
    xs1_ref[0] = x_ref[0] * dinv
    xs1_ref[1] = x_ref[1] * dinv


def _tc_b_body(w1_ref, b1_ref, w2_ref, aggp_ref, xs1_ref, dinv_ref, xs2_ref):
    dinv = dinv_ref[...]
    t0 = dinv * (aggp_ref[0, 0] + aggp_ref[1, 0] + xs1_ref[0])
    t1 = dinv * (aggp_ref[0, 1] + aggp_ref[1, 1] + xs1_ref[1])
    y0 = jnp.zeros_like(t0)
    y1 = jnp.zeros_like(t0)
    for j in range(64):
        h = jnp.maximum(t0 * w1_ref[0, j] + t1 * w1_ref[1, j] + b1_ref[j], 0.0)
        y0 = y0 + h * w2_ref[j, 0]
        y1 = y1 + h * w2_ref[j, 1]
    xs2_ref[0] = dinv * y0
    xs2_ref[1] = dinv * y1


def _tc_c_body(b2_ref, aggp_ref, xs2_ref, dinv_ref, out_ref):
    dinv = dinv_ref[...]
    out_ref[0] = dinv * (aggp_ref[0, 0] + aggp_ref[1, 0] + xs2_ref[0]) + b2_ref[0]
    out_ref[1] = dinv * (aggp_ref[0, 1] + aggp_ref[1, 1] + xs2_ref[1]) + b2_ref[1]


def _fm_spec(lead):
    if lead:
        return pl.BlockSpec((*lead, TC_BLK, 128),
                            lambda i: (*([0] * len(lead)), i, 0))
    return pl.BlockSpec((TC_BLK, 128), lambda i: (i, 0))


_SMEM = pl.BlockSpec(memory_space=pltpu.SMEM)
_FM1 = jax.ShapeDtypeStruct((FM_ROWS, 128), jnp.float32)
_FM2 = jax.ShapeDtypeStruct((2, FM_ROWS, 128), jnp.float32)

_tc_a = pl.pallas_call(
    _tc_a_body,
    grid=(TC_GRID,),
    in_specs=[_fm_spec((2,)), _fm_spec((2,))],
    out_specs=[_fm_spec(()), _fm_spec((2,))],
    out_shape=[_FM1, _FM2],
)

_tc_b = pl.pallas_call(
    _tc_b_body,
    grid=(TC_GRID,),
    in_specs=[_SMEM, _SMEM, _SMEM, _fm_spec((2, 2)), _fm_spec((2,)), _fm_spec(())],
    out_specs=[_fm_spec((2,))],
    out_shape=[_FM2],
)

_tc_c = pl.pallas_call(
    _tc_c_body,
    grid=(TC_GRID,),
    in_specs=[_SMEM, _fm_spec((2, 2)), _fm_spec((2,)), _fm_spec(())],
    out_specs=[_fm_spec((2,))],
    out_shape=[_FM2],
)


def _to_fm(flat_partials):
    return (flat_partials.reshape(2, NPAD, 2)
            .transpose(0, 2, 1)
            .reshape(2, 2, FM_ROWS, 128))


def kernel(x, edge_index, W1, b1, W2, b2):
    src = edge_index[0].astype(jnp.int32)
    dst = edge_index[1].astype(jnp.int32)
    npad_elems = TOT - 2 * E
    pad = 2 * N + (jnp.arange(npad_elems, dtype=jnp.int32) % 4096)
    src2 = jnp.concatenate(
        [jnp.stack([2 * src, 2 * src + 1], axis=1).reshape(-1), pad])
    dst2 = jnp.concatenate(
        [jnp.stack([2 * dst, 2 * dst + 1], axis=1).reshape(-1), pad])
    zeros_flat = jnp.zeros((FLAT,), jnp.float32)
    ones_flat = jnp.ones((FLAT,), jnp.float32)

    degp = _sc_agg(ones_flat, src2, dst2, zeros_flat)
    deg_fm = degp.reshape(2, NPAD, 2)[:, :, 0].reshape(2, FM_ROWS, 128)
    x_fm = jnp.pad(x, ((0, NPAD - N), (0, 0))).T.reshape(2, FM_ROWS, 128)
    dinv_fm, xs1_fm = _tc_a(deg_fm, x_fm)

    xs1_flat = xs1_fm.reshape(2, NPAD).T.reshape(FLAT)
    agg1p = _sc_agg(xs1_flat, src2, dst2, zeros_flat)
    (xs2_fm,) = _tc_b(W1, b1, W2, _to_fm(agg1p), xs1_fm, dinv_fm)

    xs2_flat = xs2_fm.reshape(2, NPAD).T.reshape(FLAT)
    agg2p = _sc_agg(xs2_flat, src2, dst2, zeros_flat)
    (out_fm,) = _tc_c(b2, _to_fm(agg2p), xs2_fm, dinv_fm)

    return out_fm.reshape(2, NPAD).T[:N]


# speedup vs baseline: 17.0920x; 1.2283x over previous
import functools

import jax
import jax.numpy as jnp
from jax import lax
from jax.experimental import pallas as pl
from jax.experimental.pallas import tpu as pltpu
from jax.experimental.pallas import tpu_sc as plsc

N = 100000
NPAD = 102400
E = 1600000
FLAT = 2 * NPAD
SLICE = FLAT // 16
IDXBUF = 2048
Q = 49
TOT = 32 * Q * IDXBUF
IL_ROWS = FLAT // 128
IL_BLK = 80
IL_GRID = IL_ROWS // IL_BLK

_mesh = plsc.VectorSubcoreMesh(
    core_axis_name="c", subcore_axis_name="s", num_cores=2, num_subcores=16
)


@functools.partial(
    pl.kernel,
    out_type=jax.ShapeDtypeStruct((2, FLAT), jnp.float32),
    mesh=_mesh,
    scratch_types=[
        pltpu.VMEM_SHARED((FLAT,), jnp.float32),
        pltpu.VMEM_SHARED((FLAT,), jnp.float32),
        pltpu.VMEM((IDXBUF,), jnp.int32),
        pltpu.VMEM((IDXBUF,), jnp.int32),
        pltpu.VMEM((IDXBUF,), jnp.float32),
        pltpu.SemaphoreType.DMA,
    ],
)
def _sc_agg(xs_hbm, src_hbm, dst_hbm, zeros_hbm, out_hbm,
            xs_sp, acc_sp, src_v, dst_v, vals_v, sem):
    cid = lax.axis_index("c")
    sid = lax.axis_index("s")
    r0 = sid * SLICE
    w = cid * 16 + sid
    pltpu.sync_copy(xs_hbm.at[pl.ds(r0, SLICE)], xs_sp.at[pl.ds(r0, SLICE)])
    pltpu.sync_copy(zeros_hbm.at[pl.ds(r0, SLICE)], acc_sp.at[pl.ds(r0, SLICE)])
    plsc.subcore_barrier()

    def body(q, carry):
        off = (w * Q + q) * IDXBUF
        pltpu.sync_copy(src_hbm.at[pl.ds(off, IDXBUF)], src_v)
        pltpu.sync_copy(dst_hbm.at[pl.ds(off, IDXBUF)], dst_v)
        pltpu.async_copy(xs_sp.at[src_v], vals_v, sem).wait()
        pltpu.sync_copy(vals_v, acc_sp.at[dst_v], add=True)
        return carry

    lax.fori_loop(0, Q, body, 0)
    plsc.subcore_barrier()
    pltpu.sync_copy(acc_sp.at[pl.ds(r0, SLICE)], out_hbm.at[cid, pl.ds(r0, SLICE)])


def _parity_masks(shape):
    lane = lax.broadcasted_iota(jnp.int32, shape, len(shape) - 1)
    mo = (lane % 2).astype(jnp.float32)
    return 1.0 - mo, mo


def _tc_a_body(degp_ref, x_ref, dinv_ref, xs1_ref):
    deg = degp_ref[0] + degp_ref[1] + 1.0
    dinv = lax.rsqrt(deg)
    dinv_ref[...] = dinv
    xs1_ref[...] = x_ref[...] * dinv


def _tc_b_body(w1_ref, b1_ref, w2_ref, aggp_ref, xs1_ref, dinv_ref, xs2_ref):
    dinv = dinv_ref[...]
    t = dinv * (aggp_ref[0] + aggp_ref[1] + xs1_ref[...])
    me, mo = _parity_masks(t.shape)
    tr = pltpu.roll(t, 127, axis=1)
    tl = pltpu.roll(t, 1, axis=1)
    t0 = me * t + mo * tl
    t1 = me * tr + mo * t
    y0 = jnp.zeros_like(t)
    y1 = jnp.zeros_like(t)
    for j in range(64):
        h = jnp.maximum(t0 * w1_ref[0, j] + t1 * w1_ref[1, j] + b1_ref[j], 0.0)
        y0 = y0 + h * w2_ref[j, 0]
        y1 = y1 + h * w2_ref[j, 1]
    xs2_ref[...] = dinv * (me * y0 + mo * y1)


def _tc_c_body(b2_ref, aggp_ref, xs2_ref, dinv_ref, out_ref):
    me, mo = _parity_masks(xs2_ref.shape)
    out_ref[...] = (dinv_ref[...] * (aggp_ref[0] + aggp_ref[1] + xs2_ref[...])
                    + me * b2_ref[0] + mo * b2_ref[1])


def _il_spec(lead):
    if lead:
        return pl.BlockSpec((*lead, IL_BLK, 128),
                            lambda i: (*([0] * len(lead)), i, 0))
    return pl.BlockSpec((IL_BLK, 128), lambda i: (i, 0))


_SMEM = pl.BlockSpec(memory_space=pltpu.SMEM)
_IL1 = jax.ShapeDtypeStruct((IL_ROWS, 128), jnp.float32)

_tc_a = pl.pallas_call(
    _tc_a_body,
    grid=(IL_GRID,),
    in_specs=[_il_spec((2,)), _il_spec(())],
    out_specs=[_il_spec(()), _il_spec(())],
    out_shape=[_IL1, _IL1],
)

_tc_b = pl.pallas_call(
    _tc_b_body,
    grid=(IL_GRID,),
    in_specs=[_SMEM, _SMEM, _SMEM, _il_spec((2,)), _il_spec(()), _il_spec(())],
    out_specs=[_il_spec(())],
    out_shape=[_IL1],
)

_tc_c = pl.pallas_call(
    _tc_c_body,
    grid=(IL_GRID,),
    in_specs=[_SMEM, _il_spec((2,)), _il_spec(()), _il_spec(())],
    out_specs=[_il_spec(())],
    out_shape=[_IL1],
)


def kernel(x, edge_index, W1, b1, W2, b2):
    src = edge_index[0].astype(jnp.int32)
    dst = edge_index[1].astype(jnp.int32)
    npad_elems = TOT - 2 * E
    pad = 2 * N + (jnp.arange(npad_elems, dtype=jnp.int32) % 4096)
    src2 = jnp.concatenate(
        [jnp.stack([2 * src, 2 * src + 1], axis=1).reshape(-1), pad])
    dst2 = jnp.concatenate(
        [jnp.stack([2 * dst, 2 * dst + 1], axis=1).reshape(-1), pad])
    zeros_flat = jnp.zeros((FLAT,), jnp.float32)
    ones_flat = jnp.ones((FLAT,), jnp.float32)

    degp = _sc_agg(ones_flat, src2, dst2, zeros_flat)
    degp_il = degp.reshape(2, IL_ROWS, 128)
    x_il = jnp.pad(x, ((0, NPAD - N), (0, 0))).reshape(IL_ROWS, 128)
    dinv_il, xs1_il = _tc_a(degp_il, x_il)

    agg1p = _sc_agg(xs1_il.reshape(FLAT), src2, dst2, zeros_flat)
    (xs2_il,) = _tc_b(W1, b1, W2, agg1p.reshape(2, IL_ROWS, 128),
                      xs1_il, dinv_il)

    agg2p = _sc_agg(xs2_il.reshape(FLAT), src2, dst2, zeros_flat)
    (out_il,) = _tc_c(b2, agg2p.reshape(2, IL_ROWS, 128), xs2_il, dinv_il)

    return out_il.reshape(NPAD, 2)[:N]
